# baseline (speedup 1.0000x reference)
import jax
import jax.numpy as jnp
from jax import lax
from jax.experimental import pallas as pl
from jax.experimental.pallas import tpu as pltpu
from jax.experimental.pallas import tpu_sc as plsc

N = 10000
E = 320000
H = 128
G = 64
C = 10

NC = 2
NS = 16
NW = NC * NS

NP = 10240
EPAD = 327680
CH = 80
TCHUNK = EPAD // CH // NS
K0 = 192
K1 = TCHUNK - K0
NSTAGE = 8
QMAX = max(K0, K1) // NSTAGE
RP = NP // NS

BR = 1024
NB = NP // BR


def _sc_agg_body(h_hbm, src_hbm, dst_hbm, zero_hbm, out_hbm, src2, dst2,
                 rows, sems_g, sems_s, agg_sh):
    c = lax.axis_index("c")
    s = lax.axis_index("s")
    row0 = pl.multiple_of(s * RP, RP)

    @pl.when(c == 0)
    def _():
        pltpu.sync_copy(h_hbm.at[pl.ds(row0, RP)], agg_sh.at[pl.ds(row0, RP)])

    @pl.when(c != 0)
    def _():
        pltpu.sync_copy(zero_hbm.at[pl.ds(row0, RP)], agg_sh.at[pl.ds(row0, RP)])

    plsc.subcore_barrier()

    def g_start(i, t):
        pltpu.async_copy(h_hbm.at[src2.at[i]], rows[t], sems_g[t])

    def g_wait(i, t):
        pltpu.make_async_copy(h_hbm.at[src2.at[i]], rows[t], sems_g[t]).wait()

    def s_start(i, t):
        pltpu.async_copy(rows[t], agg_sh.at[dst2.at[i]], sems_s[t], add=True)

    def s_wait(i, t):
        pltpu.make_async_copy(rows[t], agg_sh.at[dst2.at[i]], sems_s[t]).wait()

    def run_edges(base_chunk, k):
        if k == 0:
            return
        q = k // NSTAGE
        m = q // 4

        def body(j, carry):
            for t in range(4):
                i = 4 * j + t
                g_wait(i, t)
                s_start(i, t)
                tp = (t + 3) % 4
                if t == 0:
                    @pl.when(j > 0)
                    def _():
                        s_wait(i - 1, tp)

                    g_start(i + 3, tp)
                else:
                    s_wait(i - 1, tp)

                    @pl.when(j < m - 1)
                    def _():
                        g_start(i + 3, tp)
            return carry

        def quarter_body(quarter, carry):
            crow = pl.multiple_of(base_chunk + quarter * q, 8)
            pltpu.sync_copy(src_hbm.at[pl.ds(crow, q)], src2.at[pl.ds(0, q)])
            pltpu.sync_copy(dst_hbm.at[pl.ds(crow, q)], dst2.at[pl.ds(0, q)])
            g_start(0, 0)
            g_start(1, 1)
            g_start(2, 2)
            lax.fori_loop(0, m, body, 0)
            s_wait(q - 1, (q - 1) % 4)
            return carry

        lax.fori_loop(0, NSTAGE, quarter_body, 0)

    @pl.when(c == 0)
    def _():
        run_edges(s * TCHUNK, K0)

    @pl.when(c != 0)
    def _():
        run_edges(s * TCHUNK + K0, K1)

    plsc.subcore_barrier()
    out0 = pl.multiple_of(c * NP + s * RP, RP)
    pltpu.sync_copy(agg_sh.at[pl.ds(row0, RP)], out_hbm.at[pl.ds(out0, RP)])


def _edge_agg(h, src_arr, dst_arr, zeros_tab):
    mesh = plsc.VectorSubcoreMesh(core_axis_name="c", subcore_axis_name="s")
    fn = pl.kernel(
        _sc_agg_body,
        out_type=jax.ShapeDtypeStruct((NC * NP, H), jnp.float32),
        mesh=mesh,
        scratch_types=[
            pltpu.VMEM((QMAX, CH), jnp.int32),
            pltpu.VMEM((QMAX, CH), jnp.int32),
            [pltpu.VMEM((CH, H), jnp.float32) for _ in range(4)],
            [pltpu.SemaphoreType.DMA for _ in range(4)],
            [pltpu.SemaphoreType.DMA for _ in range(4)],
            pltpu.VMEM_SHARED((NP, H), jnp.float32),
        ],
    )
    return fn(h, src_arr, dst_arr, zeros_tab).reshape(NC, NP, H)


def _tc_mlp_body(parts_ref, w1_ref, c1_ref, w2_ref, c2_ref, batch_ref,
                 h_ref, pooled_ref):
    i = pl.program_id(0)
    s = parts_ref[0] + parts_ref[1]
    t = jnp.maximum(
        jnp.dot(s, w1_ref[...], preferred_element_type=jnp.float32) + c1_ref[...],
        0.0)
    u = jnp.maximum(
        jnp.dot(t, w2_ref[...], preferred_element_type=jnp.float32) + c2_ref[...],
        0.0)
    rowid = lax.broadcasted_iota(jnp.int32, (BR, 1), 0) + i * BR
    u = jnp.where(rowid < N, u, 0.0)
    h_ref[...] = u
    onehot = (batch_ref[0] == lax.broadcasted_iota(jnp.int32, (G, BR), 0)
              ).astype(jnp.float32)
    contrib = jnp.dot(onehot, u, preferred_element_type=jnp.float32)

    @pl.when(i == 0)
    def _():
        pooled_ref[...] = contrib

    @pl.when(i != 0)
    def _():
        pooled_ref[...] += contrib


def _mlp_and_pool(parts, w1e, c1, w2e, c2, batch_r):
    return pl.pallas_call(
        _tc_mlp_body,
        grid=(NB,),
        in_specs=[
            pl.BlockSpec((NC, BR, H), lambda i: (0, i, 0)),
            pl.BlockSpec((H, H), lambda i: (0, 0)),
            pl.BlockSpec((1, H), lambda i: (0, 0)),
            pl.BlockSpec((H, H), lambda i: (0, 0)),
            pl.BlockSpec((1, H), lambda i: (0, 0)),
            pl.BlockSpec((1, 1, BR), lambda i: (i, 0, 0)),
        ],
        out_specs=[
            pl.BlockSpec((BR, H), lambda i: (i, 0)),
            pl.BlockSpec((G, H), lambda i: (0, 0)),
        ],
        out_shape=[
            jax.ShapeDtypeStruct((NP, H), jnp.float32),
            jax.ShapeDtypeStruct((G, H), jnp.float32),
        ],
    )(parts, w1e, c1, w2e, c2, batch_r)


def _tc_head_body(p_ref, w1_ref, b1_ref, w2_ref, b2_ref, out_ref):
    hh = jnp.maximum(
        jnp.dot(p_ref[...], w1_ref[...], preferred_element_type=jnp.float32)
        + b1_ref[...], 0.0)
    out_ref[...] = (jnp.dot(hh, w2_ref[...], preferred_element_type=jnp.float32)
                    + b2_ref[...])


def _head(pooled_cat, w1, b1, w2p, b2p):
    return pl.pallas_call(
        _tc_head_body,
        out_shape=jax.ShapeDtypeStruct((G, H), jnp.float32),
    )(pooled_cat, w1, b1, w2p, b2p)


def _fold_bn(p):
    scale = 1.0 / jnp.sqrt(1.0 + 1e-5)
    a1 = p['g1'] * scale
    a2 = p['g2'] * scale
    w1e = p['W1'] * a1[None, :]
    c1 = (p['b1'] * a1 + p['be1']).reshape(1, H)
    w2e = p['W2'] * a2[None, :]
    c2 = (p['b2'] * a2 + p['be2']).reshape(1, H)
    return w1e, c1, w2e, c2


def kernel(x, edge_index, batch, params):
    zeros_tab = jnp.zeros((NP, H), jnp.float32)
    x_pad = jnp.concatenate([x, jnp.zeros((NP - N, H), jnp.float32)], axis=0)
    pad_idx = jnp.full((EPAD - E,), N, jnp.int32)
    src_arr = jnp.concatenate([edge_index[0], pad_idx]).reshape(EPAD // CH, CH)
    dst_arr = jnp.concatenate([edge_index[1], pad_idx]).reshape(EPAD // CH, CH)
    batch_r = jnp.concatenate([batch, jnp.full((NP - N,), G, jnp.int32)]
                              ).reshape(NB, 1, BR)

    h = x_pad
    pooled_list = []
    for name in ('conv0', 'conv1', 'conv2'):
        w1e, c1, w2e, c2 = _fold_bn(params[name])
        parts = _edge_agg(h, src_arr, dst_arr, zeros_tab)
        h, pooled = _mlp_and_pool(parts, w1e, c1, w2e, c2, batch_r)
        pooled_list.append(pooled)

    pooled_cat = jnp.concatenate(pooled_list, axis=1)
    lin1_b = params['lin1_b'].reshape(1, H)
    w2p = jnp.zeros((H, H), jnp.float32).at[:, :C].set(params['lin2_W'])
    b2p = jnp.zeros((1, H), jnp.float32).at[0, :C].set(params['lin2_b'])
    out_pad = _head(pooled_cat, params['lin1_W'], lin1_b, w2p, b2p)
    return out_pad[:, :C]


# speedup vs baseline: 1.0162x; 1.0162x over previous
import jax
import jax.numpy as jnp
from jax import lax
from jax.experimental import pallas as pl
from jax.experimental.pallas import tpu as pltpu
from jax.experimental.pallas import tpu_sc as plsc

N = 10000
E = 320000
H = 128
G = 64
C = 10

NC = 2
NS = 16
NW = NC * NS

NP = 10240
EPAD = 327680
CH = 80
TCHUNK = EPAD // CH // NS
K0 = 192
K1 = TCHUNK - K0
NSTAGE = 8
QMAX = max(K0, K1) // NSTAGE
RP = NP // NS

BR = 1024
NB = NP // BR


def _sc_agg_body(h_hbm, src_flat_hbm, dst_hbm, zero_hbm, out_hbm, src2, dst2,
                 rows, sems_g, sems_s, agg_sh):
    c = lax.axis_index("c")
    s = lax.axis_index("s")
    row0 = pl.multiple_of(s * RP, RP)

    @pl.when(c == 0)
    def _():
        pltpu.sync_copy(h_hbm.at[pl.ds(row0, RP)], agg_sh.at[pl.ds(row0, RP)])

    @pl.when(c != 0)
    def _():
        pltpu.sync_copy(zero_hbm.at[pl.ds(row0, RP)], agg_sh.at[pl.ds(row0, RP)])

    plsc.subcore_barrier()

    def g_start(i, t):
        pltpu.async_copy(h_hbm.at[src2.at[pl.ds(i * CH, CH)]], rows[t],
                         sems_g[t])

    def g_wait(i, t):
        pltpu.make_async_copy(h_hbm.at[src2.at[pl.ds(i * CH, CH)]], rows[t],
                              sems_g[t]).wait()

    def s_start(i, t):
        pltpu.async_copy(rows[t], agg_sh.at[dst2.at[i]], sems_s[t], add=True)

    def s_wait(i, t):
        pltpu.make_async_copy(rows[t], agg_sh.at[dst2.at[i]], sems_s[t]).wait()

    def run_edges(base_chunk, k):
        if k == 0:
            return
        q = k // NSTAGE
        m = q // 4

        def body(j, carry):
            for t in range(4):
                i = 4 * j + t
                g_wait(i, t)
                s_start(i, t)
                tp = (t + 3) % 4
                if t == 0:
                    @pl.when(j > 0)
                    def _():
                        s_wait(i - 1, tp)

                    g_start(i + 3, tp)
                else:
                    s_wait(i - 1, tp)

                    @pl.when(j < m - 1)
                    def _():
                        g_start(i + 3, tp)
            return carry

        def quarter_body(quarter, carry):
            crow = pl.multiple_of(base_chunk + quarter * q, 8)
            pltpu.sync_copy(src_flat_hbm.at[pl.ds(crow * CH, q * CH)],
                            src2.at[pl.ds(0, q * CH)])
            pltpu.sync_copy(dst_hbm.at[pl.ds(crow, q)], dst2.at[pl.ds(0, q)])
            g_start(0, 0)
            g_start(1, 1)
            g_start(2, 2)
            lax.fori_loop(0, m, body, 0)
            s_wait(q - 1, (q - 1) % 4)
            return carry

        lax.fori_loop(0, NSTAGE, quarter_body, 0)

    @pl.when(c == 0)
    def _():
        run_edges(s * TCHUNK, K0)

    @pl.when(c != 0)
    def _():
        run_edges(s * TCHUNK + K0, K1)

    plsc.subcore_barrier()
    out0 = pl.multiple_of(c * NP + s * RP, RP)
    pltpu.sync_copy(agg_sh.at[pl.ds(row0, RP)], out_hbm.at[pl.ds(out0, RP)])


def _edge_agg(h, src_arr, dst_arr, zeros_tab):
    mesh = plsc.VectorSubcoreMesh(core_axis_name="c", subcore_axis_name="s")
    fn = pl.kernel(
        _sc_agg_body,
        out_type=jax.ShapeDtypeStruct((NC * NP, H), jnp.float32),
        mesh=mesh,
        scratch_types=[
            pltpu.VMEM((QMAX * CH,), jnp.int32),
            pltpu.VMEM((QMAX, CH), jnp.int32),
            [pltpu.VMEM((CH, H), jnp.float32) for _ in range(4)],
            [pltpu.SemaphoreType.DMA for _ in range(4)],
            [pltpu.SemaphoreType.DMA for _ in range(4)],
            pltpu.VMEM_SHARED((NP, H), jnp.float32),
        ],
    )
    return fn(h, src_arr, dst_arr, zeros_tab).reshape(NC, NP, H)


def _tc_mlp_body(parts_ref, w1_ref, c1_ref, w2_ref, c2_ref, batch_ref,
                 h_ref, pooled_ref):
    i = pl.program_id(0)
    s = parts_ref[0] + parts_ref[1]
    t = jnp.maximum(
        jnp.dot(s, w1_ref[...], preferred_element_type=jnp.float32) + c1_ref[...],
        0.0)
    u = jnp.maximum(
        jnp.dot(t, w2_ref[...], preferred_element_type=jnp.float32) + c2_ref[...],
        0.0)
    rowid = lax.broadcasted_iota(jnp.int32, (BR, 1), 0) + i * BR
    u = jnp.where(rowid < N, u, 0.0)
    h_ref[...] = u
    onehot = (batch_ref[0] == lax.broadcasted_iota(jnp.int32, (G, BR), 0)
              ).astype(jnp.float32)
    contrib = jnp.dot(onehot, u, preferred_element_type=jnp.float32)

    @pl.when(i == 0)
    def _():
        pooled_ref[...] = contrib

    @pl.when(i != 0)
    def _():
        pooled_ref[...] += contrib


def _mlp_and_pool(parts, w1e, c1, w2e, c2, batch_r):
    return pl.pallas_call(
        _tc_mlp_body,
        grid=(NB,),
        in_specs=[
            pl.BlockSpec((NC, BR, H), lambda i: (0, i, 0)),
            pl.BlockSpec((H, H), lambda i: (0, 0)),
            pl.BlockSpec((1, H), lambda i: (0, 0)),
            pl.BlockSpec((H, H), lambda i: (0, 0)),
            pl.BlockSpec((1, H), lambda i: (0, 0)),
            pl.BlockSpec((1, 1, BR), lambda i: (i, 0, 0)),
        ],
        out_specs=[
            pl.BlockSpec((BR, H), lambda i: (i, 0)),
            pl.BlockSpec((G, H), lambda i: (0, 0)),
        ],
        out_shape=[
            jax.ShapeDtypeStruct((NP, H), jnp.float32),
            jax.ShapeDtypeStruct((G, H), jnp.float32),
        ],
    )(parts, w1e, c1, w2e, c2, batch_r)


def _tc_head_body(p_ref, w1_ref, b1_ref, w2_ref, b2_ref, out_ref):
    hh = jnp.maximum(
        jnp.dot(p_ref[...], w1_ref[...], preferred_element_type=jnp.float32)
        + b1_ref[...], 0.0)
    out_ref[...] = (jnp.dot(hh, w2_ref[...], preferred_element_type=jnp.float32)
                    + b2_ref[...])


def _head(pooled_cat, w1, b1, w2p, b2p):
    return pl.pallas_call(
        _tc_head_body,
        out_shape=jax.ShapeDtypeStruct((G, H), jnp.float32),
    )(pooled_cat, w1, b1, w2p, b2p)


def _fold_bn(p):
    scale = 1.0 / jnp.sqrt(1.0 + 1e-5)
    a1 = p['g1'] * scale
    a2 = p['g2'] * scale
    w1e = p['W1'] * a1[None, :]
    c1 = (p['b1'] * a1 + p['be1']).reshape(1, H)
    w2e = p['W2'] * a2[None, :]
    c2 = (p['b2'] * a2 + p['be2']).reshape(1, H)
    return w1e, c1, w2e, c2


def kernel(x, edge_index, batch, params):
    zeros_tab = jnp.zeros((NP, H), jnp.float32)
    x_pad = jnp.concatenate([x, jnp.zeros((NP - N, H), jnp.float32)], axis=0)
    pad_idx = jnp.full((EPAD - E,), N, jnp.int32)
    src_arr = jnp.concatenate([edge_index[0], pad_idx])
    dst_arr = jnp.concatenate([edge_index[1], pad_idx]).reshape(EPAD // CH, CH)
    batch_r = jnp.concatenate([batch, jnp.full((NP - N,), G, jnp.int32)]
                              ).reshape(NB, 1, BR)

    h = x_pad
    pooled_list = []
    for name in ('conv0', 'conv1', 'conv2'):
        w1e, c1, w2e, c2 = _fold_bn(params[name])
        parts = _edge_agg(h, src_arr, dst_arr, zeros_tab)
        h, pooled = _mlp_and_pool(parts, w1e, c1, w2e, c2, batch_r)
        pooled_list.append(pooled)

    pooled_cat = jnp.concatenate(pooled_list, axis=1)
    lin1_b = params['lin1_b'].reshape(1, H)
    w2p = jnp.zeros((H, H), jnp.float32).at[:, :C].set(params['lin2_W'])
    b2p = jnp.zeros((1, H), jnp.float32).at[0, :C].set(params['lin2_b'])
    out_pad = _head(pooled_cat, params['lin1_W'], lin1_b, w2p, b2p)
    return out_pad[:, :C]
